# baseline (speedup 1.0000x reference)
import jax
import jax.numpy as jnp
from jax import lax
from jax.experimental import pallas as pl

NUM_ROWS = 16384
NUM_COLS = 1000
PAD_COLS = 1024
BLOCK_ROWS = 1024


def _onehot_body(x_ref, o_ref):
    i = pl.program_id(0)
    xs = x_ref[0, pl.ds(i * BLOCK_ROWS, BLOCK_ROWS)]
    cols = lax.broadcasted_iota(jnp.int32, (BLOCK_ROWS, PAD_COLS), 1)
    o_ref[...] = (cols == xs[:, None]).astype(jnp.float32)


def kernel(x):
    x2 = x.reshape(1, NUM_ROWS).astype(jnp.int32)
    out = pl.pallas_call(
        _onehot_body,
        grid=(NUM_ROWS // BLOCK_ROWS,),
        in_specs=[pl.BlockSpec((1, NUM_ROWS), lambda i: (0, 0))],
        out_specs=pl.BlockSpec((BLOCK_ROWS, PAD_COLS), lambda i: (i, 0)),
        out_shape=jax.ShapeDtypeStruct((NUM_ROWS, PAD_COLS), jnp.float32),
    )(x2)
    return out[:, :NUM_COLS]


# speedup vs baseline: 1.0595x; 1.0595x over previous
import jax
import jax.numpy as jnp
from jax import lax
from jax.experimental import pallas as pl
from jax.experimental.pallas import tpu as pltpu

NUM_ROWS = 16384
NUM_COLS = 1000
PAD_COLS = 1024
BLOCK_ROWS = 512
NUM_SLOTS = 8
NUM_CHUNKS = NUM_ROWS // BLOCK_ROWS
NUM_ROUNDS = NUM_CHUNKS // NUM_SLOTS


def _copy(o_ref, buf_ref, sem_ref, k, ci):
    return pltpu.make_async_copy(
        buf_ref.at[k],
        o_ref.at[pl.ds(ci * BLOCK_ROWS, BLOCK_ROWS), pl.ds(0, PAD_COLS)],
        sem_ref.at[k],
    )


def _onehot_body(x_ref, o_ref, buf_ref, sem_ref):
    def one_round(r, carry):
        for k in range(NUM_SLOTS):
            ci = r * NUM_SLOTS + k

            @pl.when(r > 0)
            def _wait_prev():
                _copy(o_ref, buf_ref, sem_ref, k, ci).wait()

            xs = x_ref[0, pl.ds(ci * BLOCK_ROWS, BLOCK_ROWS)]
            cols = lax.broadcasted_iota(jnp.int32, (BLOCK_ROWS, PAD_COLS), 1)
            buf_ref[k] = (cols == xs[:, None]).astype(jnp.float32)
            _copy(o_ref, buf_ref, sem_ref, k, ci).start()
        return carry

    lax.fori_loop(0, NUM_ROUNDS, one_round, 0)
    for k in range(NUM_SLOTS):
        ci = (NUM_ROUNDS - 1) * NUM_SLOTS + k
        _copy(o_ref, buf_ref, sem_ref, k, ci).wait()


def kernel(x):
    x2 = x.reshape(1, NUM_ROWS).astype(jnp.int32)
    out = pl.pallas_call(
        _onehot_body,
        in_specs=[pl.BlockSpec(memory_space=pltpu.VMEM)],
        out_specs=pl.BlockSpec(memory_space=pl.ANY),
        out_shape=jax.ShapeDtypeStruct((NUM_ROWS, NUM_COLS), jnp.float32),
        scratch_shapes=[
            pltpu.VMEM((NUM_SLOTS, BLOCK_ROWS, PAD_COLS), jnp.float32),
            pltpu.SemaphoreType.DMA((NUM_SLOTS,)),
        ],
        compiler_params=pltpu.CompilerParams(disable_bounds_checks=True),
    )(x2)
    return out
